# baseline (speedup 1.0000x reference)
import jax
import jax.numpy as jnp
from jax.experimental import pallas as pl
from jax.experimental.pallas import tpu as pltpu

BATCH = 1024
NUM_NUMERIC = 13
NUM_CAT = 26
DEPTH = 1001
OUT_COLS = NUM_CAT * DEPTH + NUM_NUMERIC

BLOCK = 64
NBUF = 4
NBLOCKS = BATCH // BLOCK


def _body(num_ref, cat_ref, mean_ref, var_ref, out_ref, buf, sem):
    iota = jax.lax.broadcasted_iota(jnp.int32, (BLOCK, DEPTH), 1)
    inv = 1.0 / jnp.maximum(jnp.sqrt(var_ref[...]), 1e-7)
    copies = [None] * NBLOCKS
    for i in range(NBLOCKS):
        slot = i % NBUF
        if i >= NBUF:
            copies[i - NBUF].wait()
        r0 = i * BLOCK
        for f in range(NUM_CAT):
            sel = cat_ref[r0 : r0 + BLOCK, f : f + 1]
            buf[slot, :, f * DEPTH : (f + 1) * DEPTH] = (iota == sel).astype(
                jnp.float32
            )
        buf[slot, :, NUM_CAT * DEPTH :] = (
            num_ref[r0 : r0 + BLOCK, :] - mean_ref[...]
        ) * inv
        copies[i] = pltpu.make_async_copy(
            buf.at[slot], out_ref.at[pl.ds(r0, BLOCK)], sem.at[slot]
        )
        copies[i].start()
    for i in range(NBLOCKS - NBUF, NBLOCKS):
        copies[i].wait()


def kernel(numeric, cat_idx, mean, var):
    return pl.pallas_call(
        _body,
        in_specs=[
            pl.BlockSpec(memory_space=pltpu.VMEM),
            pl.BlockSpec(memory_space=pltpu.VMEM),
            pl.BlockSpec(memory_space=pltpu.VMEM),
            pl.BlockSpec(memory_space=pltpu.VMEM),
        ],
        out_specs=pl.BlockSpec(memory_space=pl.ANY),
        out_shape=jax.ShapeDtypeStruct((BATCH, OUT_COLS), jnp.float32),
        scratch_shapes=[
            pltpu.VMEM((NBUF, BLOCK, OUT_COLS), jnp.float32),
            pltpu.SemaphoreType.DMA((NBUF,)),
        ],
    )(numeric, cat_idx, mean.reshape(1, -1), var.reshape(1, -1))


# speedup vs baseline: 1.0703x; 1.0703x over previous
import jax
import jax.numpy as jnp
from jax.experimental import pallas as pl

BATCH = 1024
NUM_NUMERIC = 13
NUM_CAT = 26
DEPTH = 1001
OUT_COLS = NUM_CAT * DEPTH + NUM_NUMERIC

BLOCK = 128


def _body(num_ref, cat_ref, mean_ref, var_ref, out_ref):
    iota = jax.lax.broadcasted_iota(jnp.int32, (BLOCK, DEPTH), 1)
    for f in range(NUM_CAT):
        sel = cat_ref[:, f : f + 1]
        out_ref[:, f * DEPTH : (f + 1) * DEPTH] = (iota == sel).astype(jnp.float32)
    inv = 1.0 / jnp.maximum(jnp.sqrt(var_ref[...]), 1e-7)
    out_ref[:, NUM_CAT * DEPTH :] = (num_ref[...] - mean_ref[...]) * inv


def kernel(numeric, cat_idx, mean, var):
    grid = (BATCH // BLOCK,)
    return pl.pallas_call(
        _body,
        grid=grid,
        in_specs=[
            pl.BlockSpec((BLOCK, NUM_NUMERIC), lambda i: (i, 0)),
            pl.BlockSpec((BLOCK, NUM_CAT), lambda i: (i, 0)),
            pl.BlockSpec((1, NUM_NUMERIC), lambda i: (0, 0)),
            pl.BlockSpec((1, NUM_NUMERIC), lambda i: (0, 0)),
        ],
        out_specs=pl.BlockSpec((BLOCK, OUT_COLS), lambda i: (i, 0)),
        out_shape=jax.ShapeDtypeStruct((BATCH, OUT_COLS), jnp.float32),
    )(numeric, cat_idx, mean.reshape(1, -1), var.reshape(1, -1))
